# Initial kernel scaffold; baseline (speedup 1.0000x reference)
import jax
import jax.numpy as jnp
from jax.experimental import pallas as pl


def kernel(voxel_features, voxel_coords, p2v_map, preds, label):
    raise NotImplementedError("write your pallas kernel here")


# speedup vs baseline: 4.7496x; 4.7496x over previous
import functools

import jax
import jax.numpy as jnp
from jax import lax
from jax.experimental import pallas as pl
from jax.experimental.pallas import tpu as pltpu
from jax.experimental.pallas import tpu_sc as plsc

G = 130
G2 = G * G
GX = 132
FLAT = GX * G2
NVOX = 100_000
NPTS = 200_000
EPS = 1e-5

NC, NS = 2, 16
NW = NC * NS
L = 16
CH = 128

VOX_CH = 25
VOX_PT = VOX_CH * CH
VOX_PAD = NW * VOX_PT
VOX_ROWS = VOX_PAD // 128

PTS_CH = 50
PTS_PT = PTS_CH * CH
PTS_PAD = NW * PTS_PT

DUMMY_SCATTER = G2 * (GX - 1)
DUMMY_GATHER = G2
DUMMY_PT = NVOX + 1

_mesh = plsc.VectorSubcoreMesh(core_axis_name="c", subcore_axis_name="s")


def _wid():
    return lax.axis_index("s") * NC + lax.axis_index("c")


def _k1_scatter(grid, idx3, vals3, idx_v, val_v, sem):
    w = _wid()
    pltpu.sync_copy(idx3.at[w], idx_v)
    pltpu.sync_copy(vals3.at[w], val_v)
    cps = [
        pltpu.async_copy(val_v.at[j], grid.at[idx_v.at[j]], sem)
        for j in range(VOX_CH)
    ]
    for c in cps:
        c.wait()


_k1 = pl.kernel(
    _k1_scatter,
    mesh=_mesh,
    scratch_types=[
        pltpu.VMEM((VOX_CH, CH), jnp.int32),
        pltpu.VMEM((VOX_CH, CH), jnp.float32),
        pltpu.SemaphoreType.DMA,
    ],
)


def _k2_blur(x_ref, o_ref):
    x = x_ref[...]
    zpad = jnp.zeros(x.shape[:2] + (1,), x.dtype)
    zs = x + jnp.concatenate([x[:, :, 1:], zpad], axis=2) \
           + jnp.concatenate([zpad, x[:, :, :-1]], axis=2)
    ypad = jnp.zeros((x.shape[0], 1, x.shape[2]), x.dtype)
    o_ref[...] = zs + jnp.concatenate([zs[:, 1:, :], ypad], axis=1) \
                    + jnp.concatenate([ypad, zs[:, :-1, :]], axis=1)


_XB = 12


def _k2(grid3):
    return pl.pallas_call(
        _k2_blur,
        out_shape=jax.ShapeDtypeStruct((GX, G, G), jnp.float32),
        grid=(GX // _XB,),
        in_specs=[pl.BlockSpec((_XB, G, G), lambda i: (i, 0, 0))],
        out_specs=pl.BlockSpec((_XB, G, G), lambda i: (i, 0, 0)),
    )(grid3)


def _k3_gather(ysum, idxm, idx0, idxp, stc_out, im, i0, ip, bm, b0, bp, ov, sem):
    w = _wid()
    pltpu.sync_copy(idxm.at[w], im)
    pltpu.sync_copy(idx0.at[w], i0)
    pltpu.sync_copy(idxp.at[w], ip)
    cps = []
    for j in range(VOX_CH):
        cps.append(pltpu.async_copy(ysum.at[im.at[j]], bm.at[j], sem))
        cps.append(pltpu.async_copy(ysum.at[i0.at[j]], b0.at[j], sem))
        cps.append(pltpu.async_copy(ysum.at[ip.at[j]], bp.at[j], sem))
    for c in cps:
        c.wait()
    for j in range(VOX_CH):
        for k in range(CH // L):
            s = pl.ds(k * L, L)
            ov[j, s] = bm[j, s] + b0[j, s] + bp[j, s]
    pltpu.sync_copy(ov, stc_out.at[w])


_k3 = pl.kernel(
    _k3_gather,
    out_type=jax.ShapeDtypeStruct((NW, VOX_CH, CH), jnp.float32),
    mesh=_mesh,
    scratch_types=[
        pltpu.VMEM((VOX_CH, CH), jnp.int32),
        pltpu.VMEM((VOX_CH, CH), jnp.int32),
        pltpu.VMEM((VOX_CH, CH), jnp.int32),
        pltpu.VMEM((VOX_CH, CH), jnp.float32),
        pltpu.VMEM((VOX_CH, CH), jnp.float32),
        pltpu.VMEM((VOX_CH, CH), jnp.float32),
        pltpu.VMEM((VOX_CH, CH), jnp.float32),
        pltpu.SemaphoreType.DMA,
    ],
)


def _k4_coef(stc_ref, pr_ref, b_ref, d_ref):
    stc = stc_ref[...]
    pr = jnp.clip(pr_ref[...], 0.0, 1.0)
    rows = lax.broadcasted_iota(jnp.int32, stc.shape, 0)
    cols = lax.broadcasted_iota(jnp.int32, stc.shape, 1)
    valid = (rows * 128 + cols) < NVOX
    mean = jnp.sum(jnp.where(valid, stc, 0.0)) * (1.0 / NVOX)
    sw = 1.0 / (1.0 + jnp.exp(mean - stc))
    pos = -jnp.log(pr + EPS)
    neg = -jnp.log(1.0 - pr + EPS)
    a = sw * pos
    b = (1.0 - sw) * neg
    b_ref[...] = jnp.where(valid, b, 0.0)
    d_ref[...] = jnp.where(valid, a - b, 0.0)


def _k4(stc2, preds2):
    return pl.pallas_call(
        _k4_coef,
        out_shape=(
            jax.ShapeDtypeStruct((VOX_ROWS, 128), jnp.float32),
            jax.ShapeDtypeStruct((VOX_ROWS, 128), jnp.float32),
        ),
    )(stc2, preds2)


def _k5_loss(btab, dtab, p2v3, lab3, part_out, idx_v, lab_v, bb, db, acc_v, sem):
    w = _wid()
    pltpu.sync_copy(p2v3.at[w], idx_v)
    pltpu.sync_copy(lab3.at[w], lab_v)
    cps = []
    for j in range(PTS_CH):
        cps.append(pltpu.async_copy(btab.at[idx_v.at[j]], bb.at[j], sem))
        cps.append(pltpu.async_copy(dtab.at[idx_v.at[j]], db.at[j], sem))
    for c in cps:
        c.wait()
    acc = jnp.zeros((L,), jnp.float32)
    for j in range(PTS_CH):
        for k in range(CH // L):
            s = pl.ds(k * L, L)
            acc = acc + bb[j, s] + lab_v[j, s] * db[j, s]
    acc_v[...] = acc
    pltpu.sync_copy(acc_v, part_out.at[w])


_k5 = pl.kernel(
    _k5_loss,
    out_type=jax.ShapeDtypeStruct((NW, L), jnp.float32),
    mesh=_mesh,
    scratch_types=[
        pltpu.VMEM((PTS_CH, CH), jnp.int32),
        pltpu.VMEM((PTS_CH, CH), jnp.float32),
        pltpu.VMEM((PTS_CH, CH), jnp.float32),
        pltpu.VMEM((PTS_CH, CH), jnp.float32),
        pltpu.VMEM((L,), jnp.float32),
        pltpu.SemaphoreType.DMA,
    ],
)


@jax.jit
def kernel(voxel_features, voxel_coords, p2v_map, preds, label):
    feat = voxel_features[:, 0]
    lin = (
        (voxel_coords[:, 0] + 1) * G2
        + (voxel_coords[:, 1] + 1) * G
        + (voxel_coords[:, 2] + 1)
    ).astype(jnp.int32)

    pad_v = VOX_PAD - NVOX
    lin_s = jnp.concatenate(
        [lin, jnp.full((pad_v,), DUMMY_SCATTER, jnp.int32)]
    ).reshape(NW, VOX_CH, CH)
    feat_s = jnp.concatenate(
        [feat, jnp.zeros((pad_v,), jnp.float32)]
    ).reshape(NW, VOX_CH, CH)

    grid_ref = jax.new_ref(jnp.zeros((FLAT,), jnp.float32))
    _k1(grid_ref, lin_s, feat_s)
    grid = grid_ref[...]

    ysum = _k2(grid.reshape(GX, G, G)).reshape(FLAT)

    lin_g = jnp.concatenate(
        [lin, jnp.full((pad_v,), DUMMY_GATHER, jnp.int32)]
    ).reshape(NW, VOX_CH, CH)
    stc = _k3(ysum, lin_g - G2, lin_g, lin_g + G2)

    preds2 = jnp.concatenate(
        [preds, jnp.zeros((pad_v,), jnp.float32)]
    ).reshape(VOX_ROWS, 128)
    btab, dtab = _k4(stc.reshape(VOX_ROWS, 128), preds2)

    pad_p = PTS_PAD - NPTS
    p2v3 = jnp.concatenate(
        [p2v_map.astype(jnp.int32), jnp.full((pad_p,), DUMMY_PT, jnp.int32)]
    ).reshape(NW, PTS_CH, CH)
    lab3 = jnp.concatenate(
        [label, jnp.zeros((pad_p,), jnp.float32)]
    ).reshape(NW, PTS_CH, CH)

    parts = _k5(btab.reshape(VOX_PAD), dtab.reshape(VOX_PAD), p2v3, lab3)
    return jnp.sum(parts) * (1.0 / NPTS)


# speedup vs baseline: 12.2491x; 2.5790x over previous
import functools

import jax
import jax.numpy as jnp
from jax import lax
from jax.experimental import pallas as pl
from jax.experimental.pallas import tpu as pltpu
from jax.experimental.pallas import tpu_sc as plsc

G = 130
G2 = G * G
FLAT = G * G2
NVOX = 100_000
NPTS = 200_000
EPS = 1e-5

NC, NS = 2, 16
NW = NC * NS
L = 16
CH = 128

VOX_CH = 25
VOX_PT = VOX_CH * CH
VOX_PAD = NW * VOX_PT
VOX_ROWS = VOX_PAD // 128

PTS_CH = 50
PTS_PT = PTS_CH * CH
PTS_PAD = NW * PTS_PT

H0 = 1_098_496
H1 = FLAT - H0
SPA = 1_101_824
ZCH = 68_864
ZB = 17_216
C0 = 68_656
C1 = 68_672

SC_CH = VOX_PAD // NS // CH

DUMMY_GATHER = G2
DUMMY_PT = NVOX + 1

_mesh = plsc.VectorSubcoreMesh(core_axis_name="c", subcore_axis_name="s")


def _wid():
    return lax.axis_index("s") * NC + lax.axis_index("c")


def _k1_build(idx4, val4, grid_out, sp, idx_v, val_v, zbuf, sem):
    c = lax.axis_index("c")
    s = lax.axis_index("s")

    def _zstore(k, carry):
        zbuf[pl.ds(k * L, L)] = jnp.zeros((L,), jnp.float32)
        return carry

    lax.fori_loop(0, ZB // L, _zstore, 0)
    for r in range(ZCH // ZB):
        pltpu.sync_copy(zbuf, sp.at[pl.ds(s * ZCH + r * ZB, ZB)])
    plsc.subcore_barrier()

    pltpu.sync_copy(idx4.at[c, s], idx_v)
    pltpu.sync_copy(val4.at[c, s], val_v)
    cps = [
        pltpu.async_copy(val_v.at[j], sp.at[idx_v.at[j]], sem, add=True)
        for j in range(SC_CH)
    ]
    for cp in cps:
        cp.wait()
    plsc.subcore_barrier()

    base = s * C0
    last1 = jnp.logical_and(c == 1, s == NS - 1)

    @pl.when(jnp.logical_not(last1))
    def _():
        off = 0
        for sz in (17_168, 17_168, 17_168, 17_152):
            pltpu.sync_copy(sp.at[pl.ds(base + off, sz)], zbuf.at[pl.ds(0, sz)])
            pltpu.sync_copy(
                zbuf.at[pl.ds(0, sz)],
                grid_out.at[pl.ds(c * H0 + base + off, sz)],
            )
            off += sz

    @pl.when(last1)
    def _():
        for r in range(4):
            sz = C1 // 4
            pltpu.sync_copy(
                sp.at[pl.ds(H1 - C1 + r * sz, sz)], zbuf.at[pl.ds(0, sz)]
            )
            pltpu.sync_copy(
                zbuf.at[pl.ds(0, sz)],
                grid_out.at[pl.ds(FLAT - C1 + r * sz, sz)],
            )


_k1 = pl.kernel(
    _k1_build,
    out_type=jax.ShapeDtypeStruct((FLAT,), jnp.float32),
    mesh=_mesh,
    scratch_types=[
        pltpu.VMEM_SHARED((SPA,), jnp.float32),
        pltpu.VMEM((SC_CH, CH), jnp.int32),
        pltpu.VMEM((SC_CH, CH), jnp.float32),
        pltpu.VMEM((ZB,), jnp.float32),
        pltpu.SemaphoreType.DMA,
    ],
)


def _k2_blur(x_ref, o_ref):
    x = x_ref[...]
    zpad = jnp.zeros(x.shape[:2] + (1,), x.dtype)
    zs = x + jnp.concatenate([x[:, :, 1:], zpad], axis=2) \
           + jnp.concatenate([zpad, x[:, :, :-1]], axis=2)
    ypad = jnp.zeros((x.shape[0], 1, x.shape[2]), x.dtype)
    o_ref[...] = zs + jnp.concatenate([zs[:, 1:, :], ypad], axis=1) \
                    + jnp.concatenate([ypad, zs[:, :-1, :]], axis=1)


_XB = 10


def _k2(grid3):
    return pl.pallas_call(
        _k2_blur,
        out_shape=jax.ShapeDtypeStruct((G, G, G), jnp.float32),
        grid=(G // _XB,),
        in_specs=[pl.BlockSpec((_XB, G, G), lambda i: (i, 0, 0))],
        out_specs=pl.BlockSpec((_XB, G, G), lambda i: (i, 0, 0)),
    )(grid3)


def _k3_gather(ysum, idxm, idx0, idxp, stc_out, im, i0, ip, bm, b0, bp, ov, sem):
    w = _wid()
    pltpu.sync_copy(idxm.at[w], im)
    pltpu.sync_copy(idx0.at[w], i0)
    pltpu.sync_copy(idxp.at[w], ip)
    cps = []
    for j in range(VOX_CH):
        cps.append(pltpu.async_copy(ysum.at[im.at[j]], bm.at[j], sem))
        cps.append(pltpu.async_copy(ysum.at[i0.at[j]], b0.at[j], sem))
        cps.append(pltpu.async_copy(ysum.at[ip.at[j]], bp.at[j], sem))
    for c in cps:
        c.wait()
    for j in range(VOX_CH):
        for k in range(CH // L):
            s = pl.ds(k * L, L)
            ov[j, s] = bm[j, s] + b0[j, s] + bp[j, s]
    pltpu.sync_copy(ov, stc_out.at[w])


_k3 = pl.kernel(
    _k3_gather,
    out_type=jax.ShapeDtypeStruct((NW, VOX_CH, CH), jnp.float32),
    mesh=_mesh,
    scratch_types=[
        pltpu.VMEM((VOX_CH, CH), jnp.int32),
        pltpu.VMEM((VOX_CH, CH), jnp.int32),
        pltpu.VMEM((VOX_CH, CH), jnp.int32),
        pltpu.VMEM((VOX_CH, CH), jnp.float32),
        pltpu.VMEM((VOX_CH, CH), jnp.float32),
        pltpu.VMEM((VOX_CH, CH), jnp.float32),
        pltpu.VMEM((VOX_CH, CH), jnp.float32),
        pltpu.SemaphoreType.DMA,
    ],
)


def _k4_coef(stc_ref, pr_ref, b_ref, d_ref):
    stc = stc_ref[...]
    pr = jnp.clip(pr_ref[...], 0.0, 1.0)
    rows = lax.broadcasted_iota(jnp.int32, stc.shape, 0)
    cols = lax.broadcasted_iota(jnp.int32, stc.shape, 1)
    valid = (rows * 128 + cols) < NVOX
    mean = jnp.sum(jnp.where(valid, stc, 0.0)) * (1.0 / NVOX)
    sw = 1.0 / (1.0 + jnp.exp(mean - stc))
    pos = -jnp.log(pr + EPS)
    neg = -jnp.log(1.0 - pr + EPS)
    a = sw * pos
    b = (1.0 - sw) * neg
    b_ref[...] = jnp.where(valid, b, 0.0)
    d_ref[...] = jnp.where(valid, a - b, 0.0)


def _k4(stc2, preds2):
    return pl.pallas_call(
        _k4_coef,
        out_shape=(
            jax.ShapeDtypeStruct((VOX_ROWS, 128), jnp.float32),
            jax.ShapeDtypeStruct((VOX_ROWS, 128), jnp.float32),
        ),
    )(stc2, preds2)


def _k5_loss(btab, dtab, p2v3, lab3, part_out, idx_v, lab_v, bb, db, acc_v, sem):
    w = _wid()
    pltpu.sync_copy(p2v3.at[w], idx_v)
    pltpu.sync_copy(lab3.at[w], lab_v)
    cps = []
    for j in range(PTS_CH):
        cps.append(pltpu.async_copy(btab.at[idx_v.at[j]], bb.at[j], sem))
        cps.append(pltpu.async_copy(dtab.at[idx_v.at[j]], db.at[j], sem))
    for c in cps:
        c.wait()
    acc = jnp.zeros((L,), jnp.float32)
    for j in range(PTS_CH):
        for k in range(CH // L):
            s = pl.ds(k * L, L)
            acc = acc + bb[j, s] + lab_v[j, s] * db[j, s]
    acc_v[...] = acc
    pltpu.sync_copy(acc_v, part_out.at[w])


_k5 = pl.kernel(
    _k5_loss,
    out_type=jax.ShapeDtypeStruct((NW, L), jnp.float32),
    mesh=_mesh,
    scratch_types=[
        pltpu.VMEM((PTS_CH, CH), jnp.int32),
        pltpu.VMEM((PTS_CH, CH), jnp.float32),
        pltpu.VMEM((PTS_CH, CH), jnp.float32),
        pltpu.VMEM((PTS_CH, CH), jnp.float32),
        pltpu.VMEM((L,), jnp.float32),
        pltpu.SemaphoreType.DMA,
    ],
)


@jax.jit
def kernel(voxel_features, voxel_coords, p2v_map, preds, label):
    feat = voxel_features[:, 0]
    lin = (
        (voxel_coords[:, 0] + 1) * G2
        + (voxel_coords[:, 1] + 1) * G
        + (voxel_coords[:, 2] + 1)
    ).astype(jnp.int32)

    pad_v = VOX_PAD - NVOX
    lin_p = jnp.concatenate([lin, jnp.zeros((pad_v,), jnp.int32)])
    feat_p = jnp.concatenate([feat, jnp.zeros((pad_v,), jnp.float32)])
    lo = lin_p < H0
    idx4 = jnp.stack(
        [
            jnp.where(lo, lin_p, 0),
            jnp.where(lo, 0, lin_p - H0),
        ]
    ).reshape(NC, NS, SC_CH, CH)
    val4 = jnp.stack(
        [
            jnp.where(lo, feat_p, 0.0),
            jnp.where(lo, 0.0, feat_p),
        ]
    ).reshape(NC, NS, SC_CH, CH)

    grid = _k1(idx4, val4)

    ysum = _k2(grid.reshape(G, G, G)).reshape(FLAT)

    lin_g = jnp.concatenate(
        [lin, jnp.full((pad_v,), DUMMY_GATHER, jnp.int32)]
    ).reshape(NW, VOX_CH, CH)
    stc = _k3(ysum, lin_g - G2, lin_g, lin_g + G2)

    preds2 = jnp.concatenate(
        [preds, jnp.zeros((pad_v,), jnp.float32)]
    ).reshape(VOX_ROWS, 128)
    btab, dtab = _k4(stc.reshape(VOX_ROWS, 128), preds2)

    pad_p = PTS_PAD - NPTS
    p2v3 = jnp.concatenate(
        [p2v_map.astype(jnp.int32), jnp.full((pad_p,), DUMMY_PT, jnp.int32)]
    ).reshape(NW, PTS_CH, CH)
    lab3 = jnp.concatenate(
        [label, jnp.zeros((pad_p,), jnp.float32)]
    ).reshape(NW, PTS_CH, CH)

    parts = _k5(btab.reshape(VOX_PAD), dtab.reshape(VOX_PAD), p2v3, lab3)
    return jnp.sum(parts) * (1.0 / NPTS)


# speedup vs baseline: 15.5986x; 1.2734x over previous
import functools

import jax
import jax.numpy as jnp
from jax import lax
from jax.experimental import pallas as pl
from jax.experimental.pallas import tpu as pltpu
from jax.experimental.pallas import tpu_sc as plsc

G = 130
G2 = G * G
FLAT = G * G2
NVOX = 100_000
NPTS = 200_000
EPS = 1e-5

NC, NS = 2, 16
NW = NC * NS
L = 16
CH = 128

VOX_CH = 25
VOX_PT = VOX_CH * CH
VOX_PAD = NW * VOX_PT

PTS_CH = 50
PTS_PT = PTS_CH * CH
PTS_PAD = NW * PTS_PT

H0 = 1_098_496
H1 = FLAT - H0
SPA = 1_101_824
ZCH = 68_864
ZB = 17_216
C0 = 68_656
C1 = 68_672

SC_CH = VOX_PAD // NS // CH

DUMMY_GATHER = G2
DUMMY_PT = NVOX + 1

_mesh = plsc.VectorSubcoreMesh(core_axis_name="c", subcore_axis_name="s")


def _wid():
    return lax.axis_index("s") * NC + lax.axis_index("c")


def _k1_build(idx4, val4, grid_out, sp, idx_v, val_v, zbuf, sem, lsem):
    c = lax.axis_index("c")
    s = lax.axis_index("s")

    ld_i = pltpu.async_copy(idx4.at[c, s], idx_v, lsem)
    ld_v = pltpu.async_copy(val4.at[c, s], val_v, lsem)

    def _zstore(k, carry):
        zbuf[pl.ds(k * L, L)] = jnp.zeros((L,), jnp.float32)
        return carry

    lax.fori_loop(0, ZB // L, _zstore, 0)
    for r in range(ZCH // ZB):
        pltpu.sync_copy(zbuf, sp.at[pl.ds(s * ZCH + r * ZB, ZB)])
    ld_i.wait()
    ld_v.wait()
    plsc.subcore_barrier()

    cps = [
        pltpu.async_copy(val_v.at[j], sp.at[idx_v.at[j]], sem, add=True)
        for j in range(SC_CH)
    ]
    for cp in cps:
        cp.wait()
    plsc.subcore_barrier()

    last1 = jnp.logical_and(c == 1, s == NS - 1)

    @pl.when(jnp.logical_not(last1))
    def _():
        base = s * C0
        off = 0
        for sz in (17_168, 17_168, 17_168, 17_152):
            pltpu.sync_copy(sp.at[pl.ds(base + off, sz)], zbuf.at[pl.ds(0, sz)])
            pltpu.sync_copy(
                zbuf.at[pl.ds(0, sz)],
                grid_out.at[pl.ds(c * H0 + base + off, sz)],
            )
            off += sz

    @pl.when(last1)
    def _():
        for r in range(4):
            sz = C1 // 4
            pltpu.sync_copy(
                sp.at[pl.ds(H1 - C1 + r * sz, sz)], zbuf.at[pl.ds(0, sz)]
            )
            pltpu.sync_copy(
                zbuf.at[pl.ds(0, sz)],
                grid_out.at[pl.ds(FLAT - C1 + r * sz, sz)],
            )


_k1 = pl.kernel(
    _k1_build,
    out_type=jax.ShapeDtypeStruct((FLAT,), jnp.float32),
    mesh=_mesh,
    scratch_types=[
        pltpu.VMEM_SHARED((SPA,), jnp.float32),
        pltpu.VMEM((SC_CH, CH), jnp.int32),
        pltpu.VMEM((SC_CH, CH), jnp.float32),
        pltpu.VMEM((ZB,), jnp.float32),
        pltpu.SemaphoreType.DMA,
        pltpu.SemaphoreType.DMA,
    ],
)


def _k2_blur(x_ref, o_ref):
    a = x_ref[...]

    def tap3(v, sft):
        z = jnp.zeros((sft,), v.dtype)
        return v + jnp.concatenate([v[sft:], z]) + jnp.concatenate([z, v[:-sft]])

    o_ref[...] = tap3(tap3(tap3(a, 1), G), G2)


def _k2(grid1):
    return pl.pallas_call(
        _k2_blur,
        out_shape=jax.ShapeDtypeStruct((FLAT,), jnp.float32),
        compiler_params=pltpu.CompilerParams(
            vmem_limit_bytes=100 * 1024 * 1024
        ),
    )(grid1)


def _k3_gather(bsum, idx3, stc_out, i0, b0, ov, sem):
    w = _wid()
    pltpu.sync_copy(idx3.at[w], i0)
    cps = [
        pltpu.async_copy(bsum.at[i0.at[j]], b0.at[j], sem)
        for j in range(VOX_CH)
    ]
    for cp in cps:
        cp.wait()
    for j in range(VOX_CH):
        for k in range(CH // L):
            ov[pl.ds(j * CH + k * L, L)] = b0[j, pl.ds(k * L, L)]
    pltpu.sync_copy(ov, stc_out.at[pl.ds(w * VOX_PT, VOX_PT)])


_k3 = pl.kernel(
    _k3_gather,
    out_type=jax.ShapeDtypeStruct((VOX_PAD,), jnp.float32),
    mesh=_mesh,
    scratch_types=[
        pltpu.VMEM((VOX_CH, CH), jnp.int32),
        pltpu.VMEM((VOX_CH, CH), jnp.float32),
        pltpu.VMEM((VOX_PT,), jnp.float32),
        pltpu.SemaphoreType.DMA,
    ],
)


def _k4_coef(stc_ref, pr_ref, b_ref, d_ref):
    stc = stc_ref[...]
    pr = jnp.clip(pr_ref[...], 0.0, 1.0)
    valid = lax.broadcasted_iota(jnp.int32, stc.shape, 0) < NVOX
    mean = jnp.sum(jnp.where(valid, stc, 0.0)) * (1.0 / NVOX)
    sw = 1.0 / (1.0 + jnp.exp(mean - stc))
    pos = -jnp.log(pr + EPS)
    neg = -jnp.log(1.0 - pr + EPS)
    a = sw * pos
    b = (1.0 - sw) * neg
    b_ref[...] = jnp.where(valid, b, 0.0)
    d_ref[...] = jnp.where(valid, a - b, 0.0)


def _k4(stc1, preds1):
    return pl.pallas_call(
        _k4_coef,
        out_shape=(
            jax.ShapeDtypeStruct((VOX_PAD,), jnp.float32),
            jax.ShapeDtypeStruct((VOX_PAD,), jnp.float32),
        ),
    )(stc1, preds1)


def _k5_loss(btab, dtab, p2v3, lab3, part_out, idx_v, lab_v, bb, db,
             acc_v, sem):
    w = _wid()
    pltpu.sync_copy(p2v3.at[w], idx_v)
    pltpu.sync_copy(lab3.at[w], lab_v)
    cps = []
    for j in range(PTS_CH):
        cps.append(pltpu.async_copy(btab.at[idx_v.at[j]], bb.at[j], sem))
        cps.append(pltpu.async_copy(dtab.at[idx_v.at[j]], db.at[j], sem))
    for cp in cps:
        cp.wait()
    acc = jnp.zeros((L,), jnp.float32)
    for j in range(PTS_CH):
        for k in range(CH // L):
            sl = pl.ds(k * L, L)
            acc = acc + bb[j, sl] + lab_v[j, sl] * db[j, sl]
    acc_v[...] = acc
    pltpu.sync_copy(acc_v, part_out.at[pl.ds(w * L, L)])


_k5 = pl.kernel(
    _k5_loss,
    out_type=jax.ShapeDtypeStruct((NW * L,), jnp.float32),
    mesh=_mesh,
    scratch_types=[
        pltpu.VMEM((PTS_CH, CH), jnp.int32),
        pltpu.VMEM((PTS_CH, CH), jnp.float32),
        pltpu.VMEM((PTS_CH, CH), jnp.float32),
        pltpu.VMEM((PTS_CH, CH), jnp.float32),
        pltpu.VMEM((L,), jnp.float32),
        pltpu.SemaphoreType.DMA,
    ],
)


@jax.jit
def kernel(voxel_features, voxel_coords, p2v_map, preds, label):
    feat = voxel_features[:, 0]
    lin = (
        (voxel_coords[:, 0] + 1) * G2
        + (voxel_coords[:, 1] + 1) * G
        + (voxel_coords[:, 2] + 1)
    ).astype(jnp.int32)

    pad_v = VOX_PAD - NVOX
    lin_p = jnp.concatenate([lin, jnp.zeros((pad_v,), jnp.int32)])
    feat_p = jnp.concatenate([feat, jnp.zeros((pad_v,), jnp.float32)])
    lo = lin_p < H0
    idx4 = jnp.stack(
        [
            jnp.where(lo, lin_p, 0),
            jnp.where(lo, 0, lin_p - H0),
        ]
    ).reshape(NC, NS, SC_CH, CH)
    val4 = jnp.stack(
        [
            jnp.where(lo, feat_p, 0.0),
            jnp.where(lo, 0.0, feat_p),
        ]
    ).reshape(NC, NS, SC_CH, CH)

    grid = _k1(idx4, val4)
    bsum = _k2(grid)

    lin_g = jnp.concatenate(
        [lin, jnp.full((pad_v,), DUMMY_GATHER, jnp.int32)]
    ).reshape(NW, VOX_CH, CH)
    stc = _k3(bsum, lin_g)

    preds1 = jnp.concatenate([preds, jnp.zeros((pad_v,), jnp.float32)])
    btab, dtab = _k4(stc, preds1)

    pad_p = PTS_PAD - NPTS
    p2v3 = jnp.concatenate(
        [p2v_map.astype(jnp.int32), jnp.full((pad_p,), DUMMY_PT, jnp.int32)]
    ).reshape(NW, PTS_CH, CH)
    lab3 = jnp.concatenate(
        [label, jnp.zeros((pad_p,), jnp.float32)]
    ).reshape(NW, PTS_CH, CH)

    parts = _k5(btab, dtab, p2v3, lab3)
    return jnp.sum(parts) * (1.0 / NPTS)
